# baseline (speedup 1.0000x reference)
import functools

import jax
import jax.numpy as jnp
from jax import lax
from jax.experimental import pallas as pl
from jax.experimental.pallas import tpu as pltpu
from jax.experimental.pallas import tpu_sc as plsc


def _sc_edge_aggregate(src3, dst3, flg3, base0, tab_c, tab_r, zeros_f,
                       np_rows):
    N, F = base0.shape
    NCHUNK, _, C = src3.shape
    NW = 32
    MAXCH = (NCHUNK + NW - 1) // NW
    PAIRS = (MAXCH + 1) // 2
    NT = np_rows // 16

    mesh = plsc.VectorSubcoreMesh(core_axis_name="c", subcore_axis_name="s")

    @functools.partial(
        pl.kernel,
        mesh=mesh,
        out_type=jax.ShapeDtypeStruct((4, np_rows, F), jnp.float32),
        scratch_types=[
            pltpu.VMEM((MAXCH, 1, C), jnp.int32),
            pltpu.VMEM((MAXCH, 1, C), jnp.int32),
            pltpu.VMEM((MAXCH, 1, C), jnp.int32),
            pltpu.VMEM((2, C, F), jnp.float32),
            pltpu.VMEM_SHARED((np_rows, F), jnp.float32),
            pltpu.SemaphoreType.DMA,
            pltpu.SemaphoreType.DMA,
        ],
    )
    def k(src_h, dst_h, flg_h, tb_h, tc_h, tr_h, zf_h, out_h,
          sbuf, dbuf, fbuf, ra, acc, semA0, semA1):
        cid = lax.axis_index("c")
        sid = lax.axis_index("s")
        wid = sid * 2 + cid
        row0 = pl.multiple_of(sid * NT, 8)
        cw = (wid * NCHUNK) // NW
        cnt = ((wid + 1) * NCHUNK) // NW - cw

        pltpu.sync_copy(src_h.at[pl.ds(cw, MAXCH)], sbuf)
        pltpu.sync_copy(dst_h.at[pl.ds(cw, MAXCH)], dbuf)
        pltpu.sync_copy(flg_h.at[pl.ds(cw, MAXCH)], fbuf)

        def zero_acc():
            pltpu.sync_copy(zf_h, acc.at[pl.ds(row0, NT)])

        sems = (semA0, semA1)

        zero_acc()
        plsc.subcore_barrier()

        def issue1(i, b):
            pltpu.async_copy(tb_h.at[sbuf.at[i, 0]], ra.at[b], sems[b])

        def step1(i, b):
            @pl.when(i < cnt)
            def _():
                pltpu.make_async_copy(tb_h.at[sbuf.at[i, 0]], ra.at[b],
                                      sems[b]).wait()
                pltpu.sync_copy(ra.at[b], acc.at[dbuf.at[i, 0]], add=True)

            @pl.when(i + 2 < cnt)
            def _():
                issue1(i + 2, b)

        for b in range(2):
            @pl.when(b < cnt)
            def _():
                issue1(b, b)

        def body1(t, carry):
            step1(2 * t, 0)
            step1(2 * t + 1, 1)
            return carry

        with jax.named_scope("p1loop"):
            lax.fori_loop(0, PAIRS, body1, 0)
        plsc.subcore_barrier()
        with jax.named_scope("p1wb"):
            pltpu.sync_copy(acc.at[pl.ds(row0, NT)],
                            out_h.at[cid, pl.ds(row0, NT)])
        plsc.subcore_barrier()

        zero_acc()
        plsc.subcore_barrier()

        def body2(i, carry):
            @pl.when(i < cnt)
            def _():
                fv = fbuf[i, 0, pl.ds(0, 16)][0]
                has_c = jnp.bitwise_and(fv, 1) > 0
                has_r = jnp.bitwise_and(fv, 2) > 0

                @pl.when(has_c)
                def _():
                    pltpu.async_copy(tc_h.at[sbuf.at[i, 0]], ra.at[0], semA0)

                @pl.when(has_r)
                def _():
                    pltpu.async_copy(tr_h.at[dbuf.at[i, 0]], ra.at[1], semA1)

                @pl.when(has_c)
                def _():
                    pltpu.make_async_copy(tc_h.at[sbuf.at[i, 0]], ra.at[0],
                                          semA0).wait()
                    pltpu.sync_copy(ra.at[0], acc.at[dbuf.at[i, 0]], add=True)

                @pl.when(has_r)
                def _():
                    pltpu.make_async_copy(tr_h.at[dbuf.at[i, 0]], ra.at[1],
                                          semA1).wait()
                    pltpu.sync_copy(ra.at[1], acc.at[sbuf.at[i, 0]], add=True)

            return carry

        with jax.named_scope("p2loop"):
            lax.fori_loop(0, MAXCH, body2, 0)
        plsc.subcore_barrier()
        pltpu.sync_copy(acc.at[pl.ds(row0, NT)],
                        out_h.at[2 + cid, pl.ds(row0, NT)])

    return k(src3, dst3, flg3, base0, tab_c, tab_r, zeros_f)


def _flags_body(nj, s_ref, d_ref, p_ref, o_ref):
    s = s_ref[...]
    d = d_ref[...]
    sm = jnp.zeros(s.shape, jnp.bool_)
    dm = jnp.zeros(d.shape, jnp.bool_)
    for j in range(nj):
        pj = p_ref[0, j]
        sm = jnp.logical_or(sm, s == pj)
        dm = jnp.logical_or(dm, d == pj)
    f = (jnp.any(sm, axis=1, keepdims=True).astype(jnp.int32)
         + 2 * jnp.any(dm, axis=1, keepdims=True).astype(jnp.int32))
    o_ref[...] = jnp.broadcast_to(f, o_ref.shape)


def _chunk_flags(src2, dst2, posf):
    nchunk, c = src2.shape
    nj = posf.shape[0]
    return pl.pallas_call(
        functools.partial(_flags_body, nj),
        grid=(1,),
        in_specs=[
            pl.BlockSpec((nchunk, c), lambda i: (0, 0)),
            pl.BlockSpec((nchunk, c), lambda i: (0, 0)),
            pl.BlockSpec((1, nj), lambda i: (0, 0)),
        ],
        out_specs=pl.BlockSpec((nchunk, c), lambda i: (0, 0)),
        out_shape=jax.ShapeDtypeStruct((nchunk, c), jnp.int32),
    )(src2, dst2, posf.reshape(1, nj))

def _mmtab_body(bn, nj, x_ref, w_ref, b_ref, p_ref, o_ref, tc_ref, tr_ref):
    o_ref[...] = (
        jnp.dot(x_ref[...], w_ref[...], preferred_element_type=jnp.float32)
        + b_ref[...]
    )
    f = x_ref.shape[1]
    rowid = (pl.program_id(0) * bn
             + jax.lax.broadcasted_iota(jnp.int32, (bn, nj), 0))
    eq = (rowid == p_ref[...]).astype(jnp.float32)
    zj = jnp.zeros((bn, nj), jnp.float32)
    zr = jnp.zeros((bn, f - 2 * nj), jnp.float32)
    tc_ref[...] = jnp.concatenate([eq, zj, zr], axis=1)
    tr_ref[...] = jnp.concatenate([zj, eq, zr], axis=1)


def _mm_bias_tabs(x, w, b, posf, bn):
    n, f = x.shape
    nj = posf.shape[0]
    return pl.pallas_call(
        functools.partial(_mmtab_body, bn, nj),
        grid=(n // bn,),
        in_specs=[
            pl.BlockSpec((bn, f), lambda i: (i, 0)),
            pl.BlockSpec((f, f), lambda i: (0, 0)),
            pl.BlockSpec((1, f), lambda i: (0, 0)),
            pl.BlockSpec((1, nj), lambda i: (0, 0)),
        ],
        out_specs=[
            pl.BlockSpec((bn, f), lambda i: (i, 0)),
            pl.BlockSpec((bn, f), lambda i: (i, 0)),
            pl.BlockSpec((bn, f), lambda i: (i, 0)),
        ],
        out_shape=[
            jax.ShapeDtypeStruct((n, f), jnp.float32),
            jax.ShapeDtypeStruct((n, f), jnp.float32),
            jax.ShapeDtypeStruct((n, f), jnp.float32),
        ],
    )(x, w, b.reshape(1, f), posf.reshape(1, nj))


def _mc_body(mloc, nblocks, b_ref, ob_ref, cr_ref, tc_ref, w_ref, dvw_ref,
             u_ref, plab_ref, clab_ref, rlab_ref, rdeg_ref,
             uacc, pacc, cacc, racc, dacc):
    j = 2 * mloc
    cdims = (((0,), (0,)), ((), ()))

    @pl.when(pl.program_id(0) == 0)
    def _():
        uacc[...] = jnp.zeros_like(uacc)
        pacc[...] = jnp.zeros_like(pacc)
        cacc[...] = jnp.zeros_like(cacc)
        racc[...] = jnp.zeros_like(racc)
        dacc[...] = jnp.zeros_like(dacc)

    pre = b_ref[...] + ob_ref[0] + ob_ref[1]
    pre0w = jnp.dot(pre, w_ref[...], preferred_element_type=jnp.float32)
    cr = cr_ref[0] + cr_ref[1]
    eq = tc_ref[:, 0:j]
    cadj = cr[:, 0:j] + eq
    rm = cr[:, j:2 * j]

    for m in range(mloc):
        g = jnp.maximum(
            pre0w
            + cadj[:, 2 * m:2 * m + 1] * dvw_ref[2 * m:2 * m + 1, :]
            + cadj[:, 2 * m + 1:2 * m + 2] * dvw_ref[2 * m + 1:2 * m + 2, :],
            0.0,
        )
        uacc[pl.ds(2 * m, 2), :] += lax.dot_general(
            rm[:, 2 * m:2 * m + 2], g, cdims,
            preferred_element_type=jnp.float32)

    pacc[...] += lax.dot_general(eq, pre0w, cdims,
                                 preferred_element_type=jnp.float32)
    cacc[...] += lax.dot_general(eq, cadj, cdims,
                                 preferred_element_type=jnp.float32)
    racc[...] += lax.dot_general(eq, rm, cdims,
                                 preferred_element_type=jnp.float32)
    dacc[...] += jnp.sum(rm, axis=0, keepdims=True)

    @pl.when(pl.program_id(0) == nblocks - 1)
    def _():
        u_ref[...] = uacc[...]
        plab_ref[...] = pacc[...]
        clab_ref[...] = cacc[...]
        rlab_ref[...] = racc[...]
        rdeg_ref[...] = dacc[...]


def _merge_contract(base0, out4, tab_c, wc0, dvw, mloc, bn):
    n, f = base0.shape
    j = 2 * mloc
    nblocks = n // bn
    return pl.pallas_call(
        functools.partial(_mc_body, mloc, nblocks),
        grid=(nblocks,),
        in_specs=[
            pl.BlockSpec((bn, f), lambda i: (i, 0)),
            pl.BlockSpec((2, bn, f), lambda i: (0, i, 0)),
            pl.BlockSpec((2, bn, f), lambda i: (1, i, 0)),
            pl.BlockSpec((bn, f), lambda i: (i, 0)),
            pl.BlockSpec((f, f), lambda i: (0, 0)),
            pl.BlockSpec((j, f), lambda i: (0, 0)),
        ],
        out_specs=[
            pl.BlockSpec((j, f), lambda i: (0, 0)),
            pl.BlockSpec((j, f), lambda i: (0, 0)),
            pl.BlockSpec((j, j), lambda i: (0, 0)),
            pl.BlockSpec((j, j), lambda i: (0, 0)),
            pl.BlockSpec((1, j), lambda i: (0, 0)),
        ],
        out_shape=[
            jax.ShapeDtypeStruct((j, f), jnp.float32),
            jax.ShapeDtypeStruct((j, f), jnp.float32),
            jax.ShapeDtypeStruct((j, j), jnp.float32),
            jax.ShapeDtypeStruct((j, j), jnp.float32),
            jax.ShapeDtypeStruct((1, j), jnp.float32),
        ],
        scratch_shapes=[
            pltpu.VMEM((j, f), jnp.float32),
            pltpu.VMEM((j, f), jnp.float32),
            pltpu.VMEM((j, j), jnp.float32),
            pltpu.VMEM((j, j), jnp.float32),
            pltpu.VMEM((1, j), jnp.float32),
        ],
    )(base0, out4, out4, tab_c, wc0, dvw)


def kernel(x, edge_index, pos, Wf0_0, bf0_0, Wf1_0, bf1_0, Wc_0,
           Wf0_1, bf0_1, Wf1_1, bf1_1, Wc_1):
    N, F = x.shape
    Mloc = pos.shape[0]
    J = 2 * Mloc
    src = edge_index[0]
    dst = edge_index[1]
    posf = pos.reshape(-1)
    jar = jnp.arange(J)
    midx = jnp.repeat(jnp.arange(Mloc), 2)

    NP = ((N + 127) // 128) * 128

    dup = pos[:, 0] == pos[:, 1]
    w = jnp.stack([jnp.where(dup, 0.0, 1.0), jnp.ones((Mloc,))], axis=1
                  ).reshape(-1)

    base0, tab_c, tab_r = _mm_bias_tabs(x, Wf0_0, bf0_0, posf, bn=1000)

    lab0 = x[posf] @ Wf1_0 + bf1_0
    dval0 = lab0 - base0[posf]
    dvw = (dval0 @ Wc_0) * w[:, None]

    C = 128
    src3 = src.reshape(-1, 1, C)
    dst3 = dst.reshape(-1, 1, C)
    flags = _chunk_flags(src.reshape(-1, C), dst.reshape(-1, C), posf)
    out4 = _sc_edge_aggregate(
        src3, dst3, flags.reshape(-1, 1, C), base0, tab_c, tab_r,
        jnp.zeros((NP // 16, F), jnp.float32), NP)

    u, plab, clab, rlab, rdeg2 = _merge_contract(
        base0, out4, tab_c, Wc_0, dvw, Mloc, bn=1000)
    rdeg = rdeg2[0]

    g_arg = (plab
             + clab[jar, 2 * midx][:, None] * dvw[2 * midx]
             + clab[jar, 2 * midx + 1][:, None] * dvw[2 * midx + 1])
    glab = jax.nn.relu(g_arg)
    x1f_lab = glab @ Wf1_1 + bf1_1
    x1f_unlab_at_lab = glab @ Wf0_1 + bf0_1

    agg_main = u @ Wf0_1 + rdeg[:, None] * bf0_1
    rjk0 = rlab[2 * midx, jar]
    rjk1 = rlab[2 * midx + 1, jar]
    diff = x1f_lab - x1f_unlab_at_lab
    fix = ((w[2 * midx] * rjk0)[:, None] * diff[2 * midx]
           + (w[2 * midx + 1] * rjk1)[:, None] * diff[2 * midx + 1])

    out = jax.nn.relu((x1f_lab + agg_main + fix) @ Wc_1)
    return out.reshape(Mloc, 2, F)


# speedup vs baseline: 1.0430x; 1.0430x over previous
import functools

import jax
import jax.numpy as jnp
from jax import lax
from jax.experimental import pallas as pl
from jax.experimental.pallas import tpu as pltpu
from jax.experimental.pallas import tpu_sc as plsc


def _sc_edge_aggregate(src3, dst3, flg3, base0, tab_c, tab_r, zeros_f,
                       np_rows):
    N, F = base0.shape
    NCHUNK, _, C = src3.shape
    NW = 32
    MAXCH = (NCHUNK + NW - 1) // NW
    PAIRS = (MAXCH + 1) // 2
    NT = np_rows // 16

    mesh = plsc.VectorSubcoreMesh(core_axis_name="c", subcore_axis_name="s")

    @functools.partial(
        pl.kernel,
        mesh=mesh,
        out_type=jax.ShapeDtypeStruct((4, np_rows, F), jnp.float32),
        scratch_types=[
            pltpu.VMEM((MAXCH, 1, C), jnp.int32),
            pltpu.VMEM((MAXCH, 1, C), jnp.int32),
            pltpu.VMEM((MAXCH, 1, C), jnp.int32),
            pltpu.VMEM((2, C, F), jnp.float32),
            pltpu.VMEM_SHARED((np_rows, F), jnp.float32),
            pltpu.SemaphoreType.DMA,
            pltpu.SemaphoreType.DMA,
        ],
    )
    def k(src_h, dst_h, flg_h, tb_h, tc_h, tr_h, zf_h, out_h,
          sbuf, dbuf, fbuf, ra, acc, semA0, semA1):
        cid = lax.axis_index("c")
        sid = lax.axis_index("s")
        wid = sid * 2 + cid
        row0 = pl.multiple_of(sid * NT, 8)
        cw = (wid * NCHUNK) // NW
        cnt = ((wid + 1) * NCHUNK) // NW - cw

        pltpu.sync_copy(src_h.at[pl.ds(cw, MAXCH)], sbuf)
        pltpu.sync_copy(dst_h.at[pl.ds(cw, MAXCH)], dbuf)
        pltpu.sync_copy(flg_h.at[pl.ds(cw, MAXCH)], fbuf)

        def zero_acc():
            pltpu.sync_copy(zf_h, acc.at[pl.ds(row0, NT)])

        sems = (semA0, semA1)

        zero_acc()
        plsc.subcore_barrier()

        def issue1(i, b):
            pltpu.async_copy(tb_h.at[sbuf.at[i, 0]], ra.at[b], sems[b])

        def step1(i, b):
            @pl.when(i < cnt)
            def _():
                pltpu.make_async_copy(tb_h.at[sbuf.at[i, 0]], ra.at[b],
                                      sems[b]).wait()
                pltpu.sync_copy(ra.at[b], acc.at[dbuf.at[i, 0]], add=True)

            @pl.when(i + 2 < cnt)
            def _():
                issue1(i + 2, b)

        for b in range(2):
            @pl.when(b < cnt)
            def _():
                issue1(b, b)

        def body1(t, carry):
            step1(2 * t, 0)
            step1(2 * t + 1, 1)
            return carry

        with jax.named_scope("p1loop"):
            lax.fori_loop(0, PAIRS, body1, 0)
        plsc.subcore_barrier()
        with jax.named_scope("p1wb"):
            pltpu.sync_copy(acc.at[pl.ds(row0, NT)],
                            out_h.at[cid, pl.ds(row0, NT)])
        plsc.subcore_barrier()

        zero_acc()
        plsc.subcore_barrier()

        for bit, tab_h, gk, sk in ((1, tc_h, sbuf, dbuf),
                                   (2, tr_h, dbuf, sbuf)):
            def issue2(i, b, bit=bit, tab_h=tab_h, gk=gk):
                fv = fbuf[i, 0, pl.ds(0, 16)][0]

                @pl.when(jnp.bitwise_and(fv, bit) > 0)
                def _():
                    pltpu.async_copy(tab_h.at[gk.at[i, 0]], ra.at[b], sems[b])

            def step2(i, b, bit=bit, tab_h=tab_h, gk=gk, sk=sk,
                      issue2=issue2):
                @pl.when(i < cnt)
                def _():
                    fv = fbuf[i, 0, pl.ds(0, 16)][0]

                    @pl.when(jnp.bitwise_and(fv, bit) > 0)
                    def _():
                        pltpu.make_async_copy(tab_h.at[gk.at[i, 0]],
                                              ra.at[b], sems[b]).wait()
                        pltpu.sync_copy(ra.at[b], acc.at[sk.at[i, 0]],
                                        add=True)

                @pl.when(i + 2 < cnt)
                def _():
                    issue2(i + 2, b)

            for b in range(2):
                @pl.when(b < cnt)
                def _(b=b, issue2=issue2):
                    issue2(b, b)

            def body2(t, carry, step2=step2):
                step2(2 * t, 0)
                step2(2 * t + 1, 1)
                return carry

            with jax.named_scope("p2loop"):
                lax.fori_loop(0, PAIRS, body2, 0)
        plsc.subcore_barrier()
        pltpu.sync_copy(acc.at[pl.ds(row0, NT)],
                        out_h.at[2 + cid, pl.ds(row0, NT)])

    return k(src3, dst3, flg3, base0, tab_c, tab_r, zeros_f)


def _flags_body(nj, s_ref, d_ref, p_ref, o_ref):
    s = s_ref[...]
    d = d_ref[...]
    sm = jnp.zeros(s.shape, jnp.bool_)
    dm = jnp.zeros(d.shape, jnp.bool_)
    for j in range(nj):
        pj = p_ref[0, j]
        sm = jnp.logical_or(sm, s == pj)
        dm = jnp.logical_or(dm, d == pj)
    f = (jnp.any(sm, axis=1, keepdims=True).astype(jnp.int32)
         + 2 * jnp.any(dm, axis=1, keepdims=True).astype(jnp.int32))
    o_ref[...] = jnp.broadcast_to(f, o_ref.shape)


def _chunk_flags(src2, dst2, posf):
    nchunk, c = src2.shape
    nj = posf.shape[0]
    return pl.pallas_call(
        functools.partial(_flags_body, nj),
        grid=(1,),
        in_specs=[
            pl.BlockSpec((nchunk, c), lambda i: (0, 0)),
            pl.BlockSpec((nchunk, c), lambda i: (0, 0)),
            pl.BlockSpec((1, nj), lambda i: (0, 0)),
        ],
        out_specs=pl.BlockSpec((nchunk, c), lambda i: (0, 0)),
        out_shape=jax.ShapeDtypeStruct((nchunk, c), jnp.int32),
    )(src2, dst2, posf.reshape(1, nj))

def _mmtab_body(bn, nj, x_ref, w_ref, b_ref, p_ref, o_ref, tc_ref, tr_ref):
    o_ref[...] = (
        jnp.dot(x_ref[...], w_ref[...], preferred_element_type=jnp.float32)
        + b_ref[...]
    )
    f = x_ref.shape[1]
    rowid = (pl.program_id(0) * bn
             + jax.lax.broadcasted_iota(jnp.int32, (bn, nj), 0))
    eq = (rowid == p_ref[...]).astype(jnp.float32)
    zj = jnp.zeros((bn, nj), jnp.float32)
    zr = jnp.zeros((bn, f - 2 * nj), jnp.float32)
    tc_ref[...] = jnp.concatenate([eq, zj, zr], axis=1)
    tr_ref[...] = jnp.concatenate([zj, eq, zr], axis=1)


def _mm_bias_tabs(x, w, b, posf, bn):
    n, f = x.shape
    nj = posf.shape[0]
    return pl.pallas_call(
        functools.partial(_mmtab_body, bn, nj),
        grid=(n // bn,),
        in_specs=[
            pl.BlockSpec((bn, f), lambda i: (i, 0)),
            pl.BlockSpec((f, f), lambda i: (0, 0)),
            pl.BlockSpec((1, f), lambda i: (0, 0)),
            pl.BlockSpec((1, nj), lambda i: (0, 0)),
        ],
        out_specs=[
            pl.BlockSpec((bn, f), lambda i: (i, 0)),
            pl.BlockSpec((bn, f), lambda i: (i, 0)),
            pl.BlockSpec((bn, f), lambda i: (i, 0)),
        ],
        out_shape=[
            jax.ShapeDtypeStruct((n, f), jnp.float32),
            jax.ShapeDtypeStruct((n, f), jnp.float32),
            jax.ShapeDtypeStruct((n, f), jnp.float32),
        ],
    )(x, w, b.reshape(1, f), posf.reshape(1, nj))


def _mc_body(mloc, nblocks, b_ref, ob_ref, cr_ref, tc_ref, w_ref, dvw_ref,
             u_ref, plab_ref, clab_ref, rlab_ref, rdeg_ref,
             uacc, pacc, cacc, racc, dacc):
    j = 2 * mloc
    cdims = (((0,), (0,)), ((), ()))

    @pl.when(pl.program_id(0) == 0)
    def _():
        uacc[...] = jnp.zeros_like(uacc)
        pacc[...] = jnp.zeros_like(pacc)
        cacc[...] = jnp.zeros_like(cacc)
        racc[...] = jnp.zeros_like(racc)
        dacc[...] = jnp.zeros_like(dacc)

    pre = b_ref[...] + ob_ref[0] + ob_ref[1]
    pre0w = jnp.dot(pre, w_ref[...], preferred_element_type=jnp.float32)
    cr = cr_ref[0] + cr_ref[1]
    eq = tc_ref[:, 0:j]
    cadj = cr[:, 0:j] + eq
    rm = cr[:, j:2 * j]

    for m in range(mloc):
        g = jnp.maximum(
            pre0w
            + cadj[:, 2 * m:2 * m + 1] * dvw_ref[2 * m:2 * m + 1, :]
            + cadj[:, 2 * m + 1:2 * m + 2] * dvw_ref[2 * m + 1:2 * m + 2, :],
            0.0,
        )
        uacc[pl.ds(2 * m, 2), :] += lax.dot_general(
            rm[:, 2 * m:2 * m + 2], g, cdims,
            preferred_element_type=jnp.float32)

    pacc[...] += lax.dot_general(eq, pre0w, cdims,
                                 preferred_element_type=jnp.float32)
    cacc[...] += lax.dot_general(eq, cadj, cdims,
                                 preferred_element_type=jnp.float32)
    racc[...] += lax.dot_general(eq, rm, cdims,
                                 preferred_element_type=jnp.float32)
    dacc[...] += jnp.sum(rm, axis=0, keepdims=True)

    @pl.when(pl.program_id(0) == nblocks - 1)
    def _():
        u_ref[...] = uacc[...]
        plab_ref[...] = pacc[...]
        clab_ref[...] = cacc[...]
        rlab_ref[...] = racc[...]
        rdeg_ref[...] = dacc[...]


def _merge_contract(base0, out4, tab_c, wc0, dvw, mloc, bn):
    n, f = base0.shape
    j = 2 * mloc
    nblocks = n // bn
    return pl.pallas_call(
        functools.partial(_mc_body, mloc, nblocks),
        grid=(nblocks,),
        in_specs=[
            pl.BlockSpec((bn, f), lambda i: (i, 0)),
            pl.BlockSpec((2, bn, f), lambda i: (0, i, 0)),
            pl.BlockSpec((2, bn, f), lambda i: (1, i, 0)),
            pl.BlockSpec((bn, f), lambda i: (i, 0)),
            pl.BlockSpec((f, f), lambda i: (0, 0)),
            pl.BlockSpec((j, f), lambda i: (0, 0)),
        ],
        out_specs=[
            pl.BlockSpec((j, f), lambda i: (0, 0)),
            pl.BlockSpec((j, f), lambda i: (0, 0)),
            pl.BlockSpec((j, j), lambda i: (0, 0)),
            pl.BlockSpec((j, j), lambda i: (0, 0)),
            pl.BlockSpec((1, j), lambda i: (0, 0)),
        ],
        out_shape=[
            jax.ShapeDtypeStruct((j, f), jnp.float32),
            jax.ShapeDtypeStruct((j, f), jnp.float32),
            jax.ShapeDtypeStruct((j, j), jnp.float32),
            jax.ShapeDtypeStruct((j, j), jnp.float32),
            jax.ShapeDtypeStruct((1, j), jnp.float32),
        ],
        scratch_shapes=[
            pltpu.VMEM((j, f), jnp.float32),
            pltpu.VMEM((j, f), jnp.float32),
            pltpu.VMEM((j, j), jnp.float32),
            pltpu.VMEM((j, j), jnp.float32),
            pltpu.VMEM((1, j), jnp.float32),
        ],
    )(base0, out4, out4, tab_c, wc0, dvw)


def kernel(x, edge_index, pos, Wf0_0, bf0_0, Wf1_0, bf1_0, Wc_0,
           Wf0_1, bf0_1, Wf1_1, bf1_1, Wc_1):
    N, F = x.shape
    Mloc = pos.shape[0]
    J = 2 * Mloc
    src = edge_index[0]
    dst = edge_index[1]
    posf = pos.reshape(-1)
    jar = jnp.arange(J)
    midx = jnp.repeat(jnp.arange(Mloc), 2)

    NP = ((N + 127) // 128) * 128

    dup = pos[:, 0] == pos[:, 1]
    w = jnp.stack([jnp.where(dup, 0.0, 1.0), jnp.ones((Mloc,))], axis=1
                  ).reshape(-1)

    base0, tab_c, tab_r = _mm_bias_tabs(x, Wf0_0, bf0_0, posf, bn=1000)

    lab0 = x[posf] @ Wf1_0 + bf1_0
    dval0 = lab0 - base0[posf]
    dvw = (dval0 @ Wc_0) * w[:, None]

    C = 128
    src3 = src.reshape(-1, 1, C)
    dst3 = dst.reshape(-1, 1, C)
    flags = _chunk_flags(src.reshape(-1, C), dst.reshape(-1, C), posf)
    out4 = _sc_edge_aggregate(
        src3, dst3, flags.reshape(-1, 1, C), base0, tab_c, tab_r,
        jnp.zeros((NP // 16, F), jnp.float32), NP)

    u, plab, clab, rlab, rdeg2 = _merge_contract(
        base0, out4, tab_c, Wc_0, dvw, Mloc, bn=1000)
    rdeg = rdeg2[0]

    g_arg = (plab
             + clab[jar, 2 * midx][:, None] * dvw[2 * midx]
             + clab[jar, 2 * midx + 1][:, None] * dvw[2 * midx + 1])
    glab = jax.nn.relu(g_arg)
    x1f_lab = glab @ Wf1_1 + bf1_1
    x1f_unlab_at_lab = glab @ Wf0_1 + bf0_1

    agg_main = u @ Wf0_1 + rdeg[:, None] * bf0_1
    rjk0 = rlab[2 * midx, jar]
    rjk1 = rlab[2 * midx + 1, jar]
    diff = x1f_lab - x1f_unlab_at_lab
    fix = ((w[2 * midx] * rjk0)[:, None] * diff[2 * midx]
           + (w[2 * midx + 1] * rjk1)[:, None] * diff[2 * midx + 1])

    out = jax.nn.relu((x1f_lab + agg_main + fix) @ Wc_1)
    return out.reshape(Mloc, 2, F)
